# baseline (speedup 1.0000x reference)
import functools

import jax
import jax.numpy as jnp
from jax import lax
from jax.experimental import pallas as pl
from jax.experimental.pallas import tpu as pltpu
from jax.experimental.pallas import tpu_sc as plsc

_T, _N, _F, _E = 8, 10000, 128, 160000
_F1, _F2, _C, _H = 64, 32, 32, 64

_NC, _NS = 2, 16
_TG = 8
_TPC = _TG // _NC
_CHK = 128
_CH = 80
_EPT = _CH * _CHK
_EP = _NS * _EPT
_NACC = 10240
_ZR = _NACC // _NS

_NB = 2000
_NBUF = 8
_LOOK = 4


@functools.lru_cache(maxsize=None)
def _make_deg_kernel():
    mesh = plsc.VectorSubcoreMesh(core_axis_name="c", subcore_axis_name="s",
                                  num_cores=_NC, num_subcores=_NS)

    @functools.partial(
        pl.kernel,
        out_type=jax.ShapeDtypeStruct((_TG, _NACC, 8), jnp.float32),
        mesh=mesh,
        compiler_params=pltpu.CompilerParams(use_tc_tiling_on_sc=False),
        scratch_types=[
            pltpu.VMEM((_CH, _CHK), jnp.int32),
            pltpu.VMEM((_CHK, 8), jnp.float32),
            pltpu.VMEM_SHARED((_NACC, 8), jnp.float32),
            pltpu.SemaphoreType.DMA((_NBUF,)),
        ],
    )
    def deg_kernel(dstp_hbm, ones_hbm, zeros_hbm, out_hbm, idx_d, ones_v, acc,
                   ssem):
        c = lax.axis_index("c")
        s = lax.axis_index("s")
        pltpu.sync_copy(ones_hbm, ones_v)
        for tt in range(_TPC):
            t = c * _TPC + tt
            pltpu.sync_copy(zeros_hbm, acc.at[pl.ds(s * _ZR, _ZR)])
            plsc.subcore_barrier()
            pltpu.sync_copy(dstp_hbm.at[t, s], idx_d)

            def chunk(j, carry):
                slot = lax.rem(j, _NBUF)

                @pl.when(j >= _NBUF)
                def _():
                    pltpu.make_async_copy(ones_v, acc.at[idx_d.at[j - _NBUF]],
                                          ssem.at[slot]).wait()

                pltpu.async_copy(ones_v, acc.at[idx_d.at[j]], ssem.at[slot],
                                 add=True)
                return carry

            lax.fori_loop(0, _CH, chunk, 0)
            for k in range(_NBUF):
                j = _CH - _NBUF + k
                slot = j % _NBUF
                pltpu.make_async_copy(ones_v, acc.at[idx_d.at[j]],
                                      ssem.at[slot]).wait()
            plsc.subcore_barrier()
            pltpu.sync_copy(acc.at[pl.ds(s * _ZR, _ZR)],
                            out_hbm.at[t].at[pl.ds(s * _ZR, _ZR)])
            plsc.subcore_barrier()

    return deg_kernel


@functools.lru_cache(maxsize=None)
def _make_agg_kernel(d, nbuf, look):
    mesh = plsc.VectorSubcoreMesh(core_axis_name="c", subcore_axis_name="s",
                                  num_cores=_NC, num_subcores=_NS)

    @functools.partial(
        pl.kernel,
        out_type=jax.ShapeDtypeStruct((_TG, _NACC, d), jnp.float32),
        mesh=mesh,
        compiler_params=pltpu.CompilerParams(use_tc_tiling_on_sc=False),
        scratch_types=[
            pltpu.VMEM((_CH, _CHK), jnp.int32),
            pltpu.VMEM((_CH, _CHK), jnp.int32),
            pltpu.VMEM((nbuf, _CHK, d), jnp.float32),
            pltpu.VMEM_SHARED((_NACC, d), jnp.float32),
            pltpu.SemaphoreType.DMA((nbuf,)),
            pltpu.SemaphoreType.DMA((nbuf,)),
        ],
    )
    def agg_kernel(zn_hbm, srcp_hbm, dstp_hbm, zeros_hbm, out_hbm,
                   idx_s, idx_d, buf, acc, gsem, ssem):
        c = lax.axis_index("c")
        s = lax.axis_index("s")
        for tt in range(_TPC):
            t = c * _TPC + tt
            pltpu.sync_copy(zeros_hbm, acc.at[pl.ds(s * _ZR, _ZR)])
            plsc.subcore_barrier()
            pltpu.sync_copy(srcp_hbm.at[t, s], idx_s)
            pltpu.sync_copy(dstp_hbm.at[t, s], idx_d)

            for k in range(look):
                pltpu.async_copy(zn_hbm.at[idx_s.at[k]], buf.at[k], gsem.at[k])

            def chunk(j, carry):
                slot = lax.rem(j, nbuf)
                pltpu.make_async_copy(zn_hbm.at[idx_s.at[j]], buf.at[slot],
                                      gsem.at[slot]).wait()
                pltpu.async_copy(buf.at[slot], acc.at[idx_d.at[j]],
                                 ssem.at[slot], add=True)
                nj = j + look
                nslot = lax.rem(nj, nbuf)

                @pl.when(j >= nbuf - look)
                def _():
                    pj = nj - nbuf
                    pltpu.make_async_copy(buf.at[nslot],
                                          acc.at[idx_d.at[pj]],
                                          ssem.at[nslot]).wait()

                @pl.when(nj < _CH)
                def _():
                    pltpu.async_copy(zn_hbm.at[idx_s.at[nj]], buf.at[nslot],
                                     gsem.at[nslot])

                return carry

            lax.fori_loop(0, _CH, chunk, 0)
            for k in range(look):
                j = _CH - look + k
                slot = j % nbuf
                pltpu.make_async_copy(buf.at[slot], acc.at[idx_d.at[j]],
                                      ssem.at[slot]).wait()
            plsc.subcore_barrier()
            pltpu.sync_copy(acc.at[pl.ds(s * _ZR, _ZR)],
                            out_hbm.at[t].at[pl.ds(s * _ZR, _ZR)])
            plsc.subcore_barrier()

    return agg_kernel


def _m1_body(x_ref, d_ref, w_ref, o_ref):
    nrm = lax.rsqrt(d_ref[0][:, 0:1] + 1.0)
    z = jnp.dot(x_ref[0], w_ref[...], preferred_element_type=jnp.float32)
    o_ref[0] = z * nrm


def _matmul1(x, deg8, w):
    dout = w.shape[1]
    return pl.pallas_call(
        _m1_body,
        grid=(_TG, _N // _NB),
        in_specs=[
            pl.BlockSpec((1, _NB, _F), lambda t, j: (t, j, 0)),
            pl.BlockSpec((1, _NB, 8), lambda t, j: (t, j, 0)),
            pl.BlockSpec((_F, dout), lambda t, j: (0, 0)),
        ],
        out_specs=pl.BlockSpec((1, _NB, dout), lambda t, j: (t, j, 0)),
        out_shape=jax.ShapeDtypeStruct((_TG, _N, dout), jnp.float32),
    )(x, deg8, w)


def _mid_body(scale_out, s_ref, zn_ref, d_ref, b_ref, w_ref, o_ref):
    nrm = lax.rsqrt(d_ref[0][:, 0:1] + 1.0)
    h = jnp.maximum((s_ref[0] + zn_ref[0]) * nrm + b_ref[...], 0.0)
    z = jnp.dot(h, w_ref[...], preferred_element_type=jnp.float32)
    o_ref[0] = z * nrm if scale_out else z


def _matmul_mid(s, zn, deg8, b, w, scale_out):
    din = zn.shape[2]
    dout = w.shape[1]
    return pl.pallas_call(
        functools.partial(_mid_body, scale_out),
        grid=(_TG, _N // _NB),
        in_specs=[
            pl.BlockSpec((1, _NB, din), lambda t, j: (t, j, 0)),
            pl.BlockSpec((1, _NB, din), lambda t, j: (t, j, 0)),
            pl.BlockSpec((1, _NB, 8), lambda t, j: (t, j, 0)),
            pl.BlockSpec((1, din), lambda t, j: (0, 0)),
            pl.BlockSpec((din, dout), lambda t, j: (0, 0)),
        ],
        out_specs=pl.BlockSpec((1, _NB, dout), lambda t, j: (t, j, 0)),
        out_shape=jax.ShapeDtypeStruct((_TG, _N, dout), jnp.float32),
    )(s, zn, deg8, b.reshape(1, din), w)


def _routing_body(u_ref, v_ref):
    u = u_ref[0]
    bT = jnp.zeros((1, _N), jnp.float32)
    v = jnp.zeros((1, _C), jnp.float32)
    for _ in range(3):
        m = jnp.max(bT)
        e = jnp.exp(bT - m)
        cT = e / jnp.sum(e)
        sv = jnp.dot(cT, u, preferred_element_type=jnp.float32)
        n2 = jnp.sum(sv * sv, axis=1, keepdims=True)
        v = (n2 / (1.0 + n2)) * sv / jnp.sqrt(n2 + 1e-9)
        bT = bT + lax.dot_general(v, u, (((1,), (1,)), ((), ())),
                                  preferred_element_type=jnp.float32)
    v_ref[0] = v


def _routing(u):
    return pl.pallas_call(
        _routing_body,
        grid=(_TG,),
        in_specs=[pl.BlockSpec((1, _N, _C), lambda t: (t, 0, 0))],
        out_specs=pl.BlockSpec((1, 1, _C), lambda t: (t, 0, 0)),
        out_shape=jax.ShapeDtypeStruct((_TG, 1, _C), jnp.float32),
    )(u)


def _lstm_body(v_ref, wi1_ref, wh1_ref, b1_ref, wi2_ref, wh2_ref, b2_ref,
               wd1_ref, bd1_ref, wd2_ref, bd2_ref, g_ref, p_ref):
    h1 = jnp.zeros((1, _H), jnp.float32)
    c1 = jnp.zeros((1, _H), jnp.float32)
    h2 = jnp.zeros((1, _H), jnp.float32)
    c2 = jnp.zeros((1, _H), jnp.float32)
    for t in range(_T):
        x = v_ref[t:t + 1, :]
        z = (jnp.dot(x, wi1_ref[...], preferred_element_type=jnp.float32)
             + jnp.dot(h1, wh1_ref[...], preferred_element_type=jnp.float32)
             + b1_ref[...])
        i = jax.nn.sigmoid(z[:, :_H])
        f = jax.nn.sigmoid(z[:, _H:2 * _H])
        g = jnp.tanh(z[:, 2 * _H:3 * _H])
        o = jax.nn.sigmoid(z[:, 3 * _H:])
        c1 = f * c1 + i * g
        h1 = o * jnp.tanh(c1)
        z2 = (jnp.dot(h1, wi2_ref[...], preferred_element_type=jnp.float32)
              + jnp.dot(h2, wh2_ref[...], preferred_element_type=jnp.float32)
              + b2_ref[...])
        i2 = jax.nn.sigmoid(z2[:, :_H])
        f2 = jax.nn.sigmoid(z2[:, _H:2 * _H])
        g2 = jnp.tanh(z2[:, 2 * _H:3 * _H])
        o2 = jax.nn.sigmoid(z2[:, 3 * _H:])
        c2 = f2 * c2 + i2 * g2
        h2 = o2 * jnp.tanh(c2)
    g_ref[...] = h2
    hd = jnp.maximum(
        jnp.dot(h2, wd1_ref[...], preferred_element_type=jnp.float32)
        + bd1_ref[...], 0.0)
    p = jnp.maximum(
        jnp.dot(hd, wd2_ref[...], preferred_element_type=jnp.float32)
        + bd2_ref[...], 0.0)
    p_ref[...] = p


def _lstm_head(v, Wi1, Wh1, b1s, Wi2, Wh2, b2s, Wd1, bd1, Wd2, bd2):
    return pl.pallas_call(
        _lstm_body,
        out_shape=(jax.ShapeDtypeStruct((1, _H), jnp.float32),
                   jax.ShapeDtypeStruct((1, 1), jnp.float32)),
    )(v, Wi1, Wh1, b1s, Wi2, Wh2, b2s, Wd1, bd1, Wd2, bd2)


def kernel(features, edges, W1, b1, W2, b2, W3, b3, Wc, Wi1, Wh1, bi1, bh1,
           Wi2, Wh2, bi2, bh2, Wd1, bd1, Wd2, bd2):
    src = edges[:, 0, :]
    dst = edges[:, 1, :]
    pad = _EP - _E
    srcp = jnp.pad(src, ((0, 0), (0, pad)))
    dstp = jnp.pad(dst, ((0, 0), (0, pad)), constant_values=_N)
    srcg = srcp + ((jnp.arange(_T, dtype=jnp.int32) % _TG) * _N)[:, None]
    srcg = srcg.reshape(_T, _NS, _CH, _CHK)
    dstp = dstp.reshape(_T, _NS, _CH, _CHK)

    ones8 = jnp.ones((_CHK, 8), jnp.float32)
    zeros8 = jnp.zeros((_ZR, 8), jnp.float32)
    zeros64 = jnp.zeros((_ZR, _F1), jnp.float32)
    zeros32 = jnp.zeros((_ZR, _C), jnp.float32)

    ngrp = _T // _TG
    sg = [srcg[h * _TG:(h + 1) * _TG] for h in range(ngrp)]
    dg = [dstp[h * _TG:(h + 1) * _TG] for h in range(ngrp)]
    fg = [features[h * _TG:(h + 1) * _TG] for h in range(ngrp)]

    deg8 = [_make_deg_kernel()(dg[h], ones8, zeros8) for h in range(ngrp)]

    zn1, s1, zn2, s2, zn3, s3, u, v = ([None] * ngrp for _ in range(8))
    for h in range(ngrp):
        zn1[h] = _matmul1(fg[h], deg8[h], W1)
        s1[h] = _make_agg_kernel(_F1, 8, 4)(
            zn1[h].reshape(_TG * _N, _F1), sg[h], dg[h], zeros64)
    for h in range(ngrp):
        zn2[h] = _matmul_mid(s1[h], zn1[h], deg8[h], b1, W2, True)
        s2[h] = _make_agg_kernel(_C, 8, 4)(
            zn2[h].reshape(_TG * _N, _C), sg[h], dg[h], zeros32)
    for h in range(ngrp):
        zn3[h] = _matmul_mid(s2[h], zn2[h], deg8[h], b2, W3, True)
        s3[h] = _make_agg_kernel(_C, 8, 4)(
            zn3[h].reshape(_TG * _N, _C), sg[h], dg[h], zeros32)
    for h in range(ngrp):
        u[h] = _matmul_mid(s3[h], zn3[h], deg8[h], b3, Wc, False)
        v[h] = _routing(u[h])

    v = jnp.concatenate(v).reshape(_T, _C)

    graph_rep, p = _lstm_head(
        v, Wi1, Wh1, (bi1 + bh1).reshape(1, 4 * _H),
        Wi2, Wh2, (bi2 + bh2).reshape(1, 4 * _H),
        Wd1, bd1.reshape(1, 32), Wd2, bd2.reshape(1, 1))
    return graph_rep, p.reshape((1,))


# speedup vs baseline: 1.0170x; 1.0170x over previous
import functools

import jax
import jax.numpy as jnp
from jax import lax
from jax.experimental import pallas as pl
from jax.experimental.pallas import tpu as pltpu
from jax.experimental.pallas import tpu_sc as plsc

_T, _N, _F, _E = 8, 10000, 128, 160000
_F1, _F2, _C, _H = 64, 32, 32, 64

_NC, _NS = 2, 16
_TG = 2
_TPC = _TG // _NC
_CHK = 128
_CH = 80
_EPT = _CH * _CHK
_EP = _NS * _EPT
_NACC = 10240
_ZR = _NACC // _NS

_NB = 2000
_NBUF = 8
_LOOK = 4


@functools.lru_cache(maxsize=None)
def _make_deg_kernel():
    mesh = plsc.VectorSubcoreMesh(core_axis_name="c", subcore_axis_name="s",
                                  num_cores=_NC, num_subcores=_NS)

    @functools.partial(
        pl.kernel,
        out_type=jax.ShapeDtypeStruct((_TG, _NACC, 8), jnp.float32),
        mesh=mesh,
        compiler_params=pltpu.CompilerParams(use_tc_tiling_on_sc=False),
        scratch_types=[
            pltpu.VMEM((_CH, _CHK), jnp.int32),
            pltpu.VMEM((_CHK, 8), jnp.float32),
            pltpu.VMEM_SHARED((_NACC, 8), jnp.float32),
            pltpu.SemaphoreType.DMA((_NBUF,)),
        ],
    )
    def deg_kernel(dstp_hbm, ones_hbm, zeros_hbm, out_hbm, idx_d, ones_v, acc,
                   ssem):
        c = lax.axis_index("c")
        s = lax.axis_index("s")
        pltpu.sync_copy(ones_hbm, ones_v)
        for tt in range(_TPC):
            t = c * _TPC + tt
            pltpu.sync_copy(zeros_hbm, acc.at[pl.ds(s * _ZR, _ZR)])
            plsc.subcore_barrier()
            pltpu.sync_copy(dstp_hbm.at[t, s], idx_d)

            def chunk(j, carry):
                slot = lax.rem(j, _NBUF)

                @pl.when(j >= _NBUF)
                def _():
                    pltpu.make_async_copy(ones_v, acc.at[idx_d.at[j - _NBUF]],
                                          ssem.at[slot]).wait()

                pltpu.async_copy(ones_v, acc.at[idx_d.at[j]], ssem.at[slot],
                                 add=True)
                return carry

            lax.fori_loop(0, _CH, chunk, 0)
            for k in range(_NBUF):
                j = _CH - _NBUF + k
                slot = j % _NBUF
                pltpu.make_async_copy(ones_v, acc.at[idx_d.at[j]],
                                      ssem.at[slot]).wait()
            plsc.subcore_barrier()
            pltpu.sync_copy(acc.at[pl.ds(s * _ZR, _ZR)],
                            out_hbm.at[t].at[pl.ds(s * _ZR, _ZR)])
            plsc.subcore_barrier()

    return deg_kernel


@functools.lru_cache(maxsize=None)
def _make_agg_kernel(d, nbuf, look):
    mesh = plsc.VectorSubcoreMesh(core_axis_name="c", subcore_axis_name="s",
                                  num_cores=_NC, num_subcores=_NS)

    @functools.partial(
        pl.kernel,
        out_type=jax.ShapeDtypeStruct((_TG, _NACC, d), jnp.float32),
        mesh=mesh,
        compiler_params=pltpu.CompilerParams(use_tc_tiling_on_sc=False),
        scratch_types=[
            pltpu.VMEM((_CH, _CHK), jnp.int32),
            pltpu.VMEM((_CH, _CHK), jnp.int32),
            pltpu.VMEM((nbuf, _CHK, d), jnp.float32),
            pltpu.VMEM_SHARED((_NACC, d), jnp.float32),
            pltpu.SemaphoreType.DMA((nbuf,)),
            pltpu.SemaphoreType.DMA((nbuf,)),
        ],
    )
    def agg_kernel(zn_hbm, srcp_hbm, dstp_hbm, zeros_hbm, out_hbm,
                   idx_s, idx_d, buf, acc, gsem, ssem):
        c = lax.axis_index("c")
        s = lax.axis_index("s")
        for tt in range(_TPC):
            t = c * _TPC + tt
            pltpu.sync_copy(zeros_hbm, acc.at[pl.ds(s * _ZR, _ZR)])
            plsc.subcore_barrier()
            pltpu.sync_copy(srcp_hbm.at[t, s], idx_s)
            pltpu.sync_copy(dstp_hbm.at[t, s], idx_d)

            for k in range(look):
                pltpu.async_copy(zn_hbm.at[idx_s.at[k]], buf.at[k], gsem.at[k])

            def chunk(j, carry):
                slot = lax.rem(j, nbuf)
                pltpu.make_async_copy(zn_hbm.at[idx_s.at[j]], buf.at[slot],
                                      gsem.at[slot]).wait()
                pltpu.async_copy(buf.at[slot], acc.at[idx_d.at[j]],
                                 ssem.at[slot], add=True)
                nj = j + look
                nslot = lax.rem(nj, nbuf)

                @pl.when(j >= nbuf - look)
                def _():
                    pj = nj - nbuf
                    pltpu.make_async_copy(buf.at[nslot],
                                          acc.at[idx_d.at[pj]],
                                          ssem.at[nslot]).wait()

                @pl.when(nj < _CH)
                def _():
                    pltpu.async_copy(zn_hbm.at[idx_s.at[nj]], buf.at[nslot],
                                     gsem.at[nslot])

                return carry

            lax.fori_loop(0, _CH, chunk, 0)
            for k in range(look):
                j = _CH - look + k
                slot = j % nbuf
                pltpu.make_async_copy(buf.at[slot], acc.at[idx_d.at[j]],
                                      ssem.at[slot]).wait()
            plsc.subcore_barrier()
            pltpu.sync_copy(acc.at[pl.ds(s * _ZR, _ZR)],
                            out_hbm.at[t].at[pl.ds(s * _ZR, _ZR)])
            plsc.subcore_barrier()

    return agg_kernel


def _m1_body(x_ref, d_ref, w_ref, o_ref):
    nrm = lax.rsqrt(d_ref[0][:, 0:1] + 1.0)
    z = jnp.dot(x_ref[0], w_ref[...], preferred_element_type=jnp.float32)
    o_ref[0] = z * nrm


def _matmul1(x, deg8, w):
    dout = w.shape[1]
    return pl.pallas_call(
        _m1_body,
        grid=(_TG, _N // _NB),
        in_specs=[
            pl.BlockSpec((1, _NB, _F), lambda t, j: (t, j, 0)),
            pl.BlockSpec((1, _NB, 8), lambda t, j: (t, j, 0)),
            pl.BlockSpec((_F, dout), lambda t, j: (0, 0)),
        ],
        out_specs=pl.BlockSpec((1, _NB, dout), lambda t, j: (t, j, 0)),
        out_shape=jax.ShapeDtypeStruct((_TG, _N, dout), jnp.float32),
    )(x, deg8, w)


def _mid_body(scale_out, s_ref, zn_ref, d_ref, b_ref, w_ref, o_ref):
    nrm = lax.rsqrt(d_ref[0][:, 0:1] + 1.0)
    h = jnp.maximum((s_ref[0] + zn_ref[0]) * nrm + b_ref[...], 0.0)
    z = jnp.dot(h, w_ref[...], preferred_element_type=jnp.float32)
    o_ref[0] = z * nrm if scale_out else z


def _matmul_mid(s, zn, deg8, b, w, scale_out):
    din = zn.shape[2]
    dout = w.shape[1]
    return pl.pallas_call(
        functools.partial(_mid_body, scale_out),
        grid=(_TG, _N // _NB),
        in_specs=[
            pl.BlockSpec((1, _NB, din), lambda t, j: (t, j, 0)),
            pl.BlockSpec((1, _NB, din), lambda t, j: (t, j, 0)),
            pl.BlockSpec((1, _NB, 8), lambda t, j: (t, j, 0)),
            pl.BlockSpec((1, din), lambda t, j: (0, 0)),
            pl.BlockSpec((din, dout), lambda t, j: (0, 0)),
        ],
        out_specs=pl.BlockSpec((1, _NB, dout), lambda t, j: (t, j, 0)),
        out_shape=jax.ShapeDtypeStruct((_TG, _N, dout), jnp.float32),
    )(s, zn, deg8, b.reshape(1, din), w)


def _routing_body(u_ref, v_ref):
    u = u_ref[0]
    bT = jnp.zeros((1, _N), jnp.float32)
    v = jnp.zeros((1, _C), jnp.float32)
    for _ in range(3):
        m = jnp.max(bT)
        e = jnp.exp(bT - m)
        cT = e / jnp.sum(e)
        sv = jnp.dot(cT, u, preferred_element_type=jnp.float32)
        n2 = jnp.sum(sv * sv, axis=1, keepdims=True)
        v = (n2 / (1.0 + n2)) * sv / jnp.sqrt(n2 + 1e-9)
        bT = bT + lax.dot_general(v, u, (((1,), (1,)), ((), ())),
                                  preferred_element_type=jnp.float32)
    v_ref[0] = v


def _routing(u):
    return pl.pallas_call(
        _routing_body,
        grid=(_TG,),
        in_specs=[pl.BlockSpec((1, _N, _C), lambda t: (t, 0, 0))],
        out_specs=pl.BlockSpec((1, 1, _C), lambda t: (t, 0, 0)),
        out_shape=jax.ShapeDtypeStruct((_TG, 1, _C), jnp.float32),
    )(u)


def _lstm_body(v_ref, wi1_ref, wh1_ref, b1_ref, wi2_ref, wh2_ref, b2_ref,
               wd1_ref, bd1_ref, wd2_ref, bd2_ref, g_ref, p_ref):
    h1 = jnp.zeros((1, _H), jnp.float32)
    c1 = jnp.zeros((1, _H), jnp.float32)
    h2 = jnp.zeros((1, _H), jnp.float32)
    c2 = jnp.zeros((1, _H), jnp.float32)
    for t in range(_T):
        x = v_ref[t:t + 1, :]
        z = (jnp.dot(x, wi1_ref[...], preferred_element_type=jnp.float32)
             + jnp.dot(h1, wh1_ref[...], preferred_element_type=jnp.float32)
             + b1_ref[...])
        i = jax.nn.sigmoid(z[:, :_H])
        f = jax.nn.sigmoid(z[:, _H:2 * _H])
        g = jnp.tanh(z[:, 2 * _H:3 * _H])
        o = jax.nn.sigmoid(z[:, 3 * _H:])
        c1 = f * c1 + i * g
        h1 = o * jnp.tanh(c1)
        z2 = (jnp.dot(h1, wi2_ref[...], preferred_element_type=jnp.float32)
              + jnp.dot(h2, wh2_ref[...], preferred_element_type=jnp.float32)
              + b2_ref[...])
        i2 = jax.nn.sigmoid(z2[:, :_H])
        f2 = jax.nn.sigmoid(z2[:, _H:2 * _H])
        g2 = jnp.tanh(z2[:, 2 * _H:3 * _H])
        o2 = jax.nn.sigmoid(z2[:, 3 * _H:])
        c2 = f2 * c2 + i2 * g2
        h2 = o2 * jnp.tanh(c2)
    g_ref[...] = h2
    hd = jnp.maximum(
        jnp.dot(h2, wd1_ref[...], preferred_element_type=jnp.float32)
        + bd1_ref[...], 0.0)
    p = jnp.maximum(
        jnp.dot(hd, wd2_ref[...], preferred_element_type=jnp.float32)
        + bd2_ref[...], 0.0)
    p_ref[...] = p


def _lstm_head(v, Wi1, Wh1, b1s, Wi2, Wh2, b2s, Wd1, bd1, Wd2, bd2):
    return pl.pallas_call(
        _lstm_body,
        out_shape=(jax.ShapeDtypeStruct((1, _H), jnp.float32),
                   jax.ShapeDtypeStruct((1, 1), jnp.float32)),
    )(v, Wi1, Wh1, b1s, Wi2, Wh2, b2s, Wd1, bd1, Wd2, bd2)


def kernel(features, edges, W1, b1, W2, b2, W3, b3, Wc, Wi1, Wh1, bi1, bh1,
           Wi2, Wh2, bi2, bh2, Wd1, bd1, Wd2, bd2):
    src = edges[:, 0, :]
    dst = edges[:, 1, :]
    pad = _EP - _E
    srcp = jnp.pad(src, ((0, 0), (0, pad)))
    dstp = jnp.pad(dst, ((0, 0), (0, pad)), constant_values=_N)
    srcg = srcp + ((jnp.arange(_T, dtype=jnp.int32) % _TG) * _N)[:, None]
    srcg = srcg.reshape(_T, _NS, _CH, _CHK)
    dstp = dstp.reshape(_T, _NS, _CH, _CHK)

    ones8 = jnp.ones((_CHK, 8), jnp.float32)
    zeros8 = jnp.zeros((_ZR, 8), jnp.float32)
    zeros64 = jnp.zeros((_ZR, _F1), jnp.float32)
    zeros32 = jnp.zeros((_ZR, _C), jnp.float32)

    ngrp = _T // _TG
    sg = [srcg[h * _TG:(h + 1) * _TG] for h in range(ngrp)]
    dg = [dstp[h * _TG:(h + 1) * _TG] for h in range(ngrp)]
    fg = [features[h * _TG:(h + 1) * _TG] for h in range(ngrp)]

    deg8 = [_make_deg_kernel()(dg[h], ones8, zeros8) for h in range(ngrp)]

    zn1, s1, zn2, s2, zn3, s3, u, v = ([None] * ngrp for _ in range(8))
    for h in range(ngrp):
        zn1[h] = _matmul1(fg[h], deg8[h], W1)
        s1[h] = _make_agg_kernel(_F1, 8, 4)(
            zn1[h].reshape(_TG * _N, _F1), sg[h], dg[h], zeros64)
    for h in range(ngrp):
        zn2[h] = _matmul_mid(s1[h], zn1[h], deg8[h], b1, W2, True)
        s2[h] = _make_agg_kernel(_C, 8, 4)(
            zn2[h].reshape(_TG * _N, _C), sg[h], dg[h], zeros32)
    for h in range(ngrp):
        zn3[h] = _matmul_mid(s2[h], zn2[h], deg8[h], b2, W3, True)
        s3[h] = _make_agg_kernel(_C, 8, 4)(
            zn3[h].reshape(_TG * _N, _C), sg[h], dg[h], zeros32)
    for h in range(ngrp):
        u[h] = _matmul_mid(s3[h], zn3[h], deg8[h], b3, Wc, False)
        v[h] = _routing(u[h])

    v = jnp.concatenate(v).reshape(_T, _C)

    graph_rep, p = _lstm_head(
        v, Wi1, Wh1, (bi1 + bh1).reshape(1, 4 * _H),
        Wi2, Wh2, (bi2 + bh2).reshape(1, 4 * _H),
        Wd1, bd1.reshape(1, 32), Wd2, bd2.reshape(1, 1))
    return graph_rep, p.reshape((1,))


# speedup vs baseline: 1.0575x; 1.0398x over previous
import functools

import jax
import jax.numpy as jnp
from jax import lax
from jax.experimental import pallas as pl
from jax.experimental.pallas import tpu as pltpu
from jax.experimental.pallas import tpu_sc as plsc

_T, _N, _F, _E = 8, 10000, 128, 160000
_F1, _F2, _C, _H = 64, 32, 32, 64

_NC, _NS = 2, 16
_TG = 4
_TPC = _TG // _NC
_CHK = 128
_CH = 80
_EPT = _CH * _CHK
_EP = _NS * _EPT
_NACC = 10240
_ZR = _NACC // _NS

_NB = 2000
_NBUF = 8
_LOOK = 4


@functools.lru_cache(maxsize=None)
def _make_deg_kernel():
    mesh = plsc.VectorSubcoreMesh(core_axis_name="c", subcore_axis_name="s",
                                  num_cores=_NC, num_subcores=_NS)

    @functools.partial(
        pl.kernel,
        out_type=jax.ShapeDtypeStruct((_TG, _NACC, 8), jnp.float32),
        mesh=mesh,
        compiler_params=pltpu.CompilerParams(use_tc_tiling_on_sc=False),
        scratch_types=[
            pltpu.VMEM((_CH, _CHK), jnp.int32),
            pltpu.VMEM((_CHK, 8), jnp.float32),
            pltpu.VMEM_SHARED((_NACC, 8), jnp.float32),
            pltpu.SemaphoreType.DMA((_NBUF,)),
        ],
    )
    def deg_kernel(dstp_hbm, ones_hbm, zeros_hbm, out_hbm, idx_d, ones_v, acc,
                   ssem):
        c = lax.axis_index("c")
        s = lax.axis_index("s")
        pltpu.sync_copy(ones_hbm, ones_v)
        for tt in range(_TPC):
            t = c * _TPC + tt
            pltpu.sync_copy(zeros_hbm, acc.at[pl.ds(s * _ZR, _ZR)])
            plsc.subcore_barrier()
            pltpu.sync_copy(dstp_hbm.at[t, s], idx_d)

            def chunk(j, carry):
                slot = lax.rem(j, _NBUF)

                @pl.when(j >= _NBUF)
                def _():
                    pltpu.make_async_copy(ones_v, acc.at[idx_d.at[j - _NBUF]],
                                          ssem.at[slot]).wait()

                pltpu.async_copy(ones_v, acc.at[idx_d.at[j]], ssem.at[slot],
                                 add=True)
                return carry

            lax.fori_loop(0, _CH, chunk, 0)
            for k in range(_NBUF):
                j = _CH - _NBUF + k
                slot = j % _NBUF
                pltpu.make_async_copy(ones_v, acc.at[idx_d.at[j]],
                                      ssem.at[slot]).wait()
            plsc.subcore_barrier()
            pltpu.sync_copy(acc.at[pl.ds(s * _ZR, _ZR)],
                            out_hbm.at[t].at[pl.ds(s * _ZR, _ZR)])
            plsc.subcore_barrier()

    return deg_kernel


@functools.lru_cache(maxsize=None)
def _make_agg_kernel(d, nbuf, look):
    mesh = plsc.VectorSubcoreMesh(core_axis_name="c", subcore_axis_name="s",
                                  num_cores=_NC, num_subcores=_NS)

    @functools.partial(
        pl.kernel,
        out_type=jax.ShapeDtypeStruct((_TG, _NACC, d), jnp.float32),
        mesh=mesh,
        compiler_params=pltpu.CompilerParams(use_tc_tiling_on_sc=False),
        scratch_types=[
            pltpu.VMEM((_CH, _CHK), jnp.int32),
            pltpu.VMEM((_CH, _CHK), jnp.int32),
            pltpu.VMEM((nbuf, _CHK, d), jnp.float32),
            pltpu.VMEM_SHARED((_NACC, d), jnp.float32),
            pltpu.SemaphoreType.DMA((nbuf,)),
            pltpu.SemaphoreType.DMA((nbuf,)),
        ],
    )
    def agg_kernel(zn_hbm, srcp_hbm, dstp_hbm, zeros_hbm, out_hbm,
                   idx_s, idx_d, buf, acc, gsem, ssem):
        c = lax.axis_index("c")
        s = lax.axis_index("s")
        for tt in range(_TPC):
            t = c * _TPC + tt
            pltpu.sync_copy(zeros_hbm, acc.at[pl.ds(s * _ZR, _ZR)])
            plsc.subcore_barrier()
            pltpu.sync_copy(srcp_hbm.at[t, s], idx_s)
            pltpu.sync_copy(dstp_hbm.at[t, s], idx_d)

            for k in range(look):
                pltpu.async_copy(zn_hbm.at[idx_s.at[k]], buf.at[k], gsem.at[k])

            def chunk(j, carry):
                slot = lax.rem(j, nbuf)
                pltpu.make_async_copy(zn_hbm.at[idx_s.at[j]], buf.at[slot],
                                      gsem.at[slot]).wait()
                pltpu.async_copy(buf.at[slot], acc.at[idx_d.at[j]],
                                 ssem.at[slot], add=True)
                nj = j + look
                nslot = lax.rem(nj, nbuf)

                @pl.when(j >= nbuf - look)
                def _():
                    pj = nj - nbuf
                    pltpu.make_async_copy(buf.at[nslot],
                                          acc.at[idx_d.at[pj]],
                                          ssem.at[nslot]).wait()

                @pl.when(nj < _CH)
                def _():
                    pltpu.async_copy(zn_hbm.at[idx_s.at[nj]], buf.at[nslot],
                                     gsem.at[nslot])

                return carry

            lax.fori_loop(0, _CH, chunk, 0)
            for k in range(look):
                j = _CH - look + k
                slot = j % nbuf
                pltpu.make_async_copy(buf.at[slot], acc.at[idx_d.at[j]],
                                      ssem.at[slot]).wait()
            plsc.subcore_barrier()
            pltpu.sync_copy(acc.at[pl.ds(s * _ZR, _ZR)],
                            out_hbm.at[t].at[pl.ds(s * _ZR, _ZR)])
            plsc.subcore_barrier()

    return agg_kernel


def _m1_body(x_ref, d_ref, w_ref, o_ref):
    nrm = lax.rsqrt(d_ref[0][:, 0:1] + 1.0)
    z = jnp.dot(x_ref[0], w_ref[...], preferred_element_type=jnp.float32)
    o_ref[0] = z * nrm


def _matmul1(x, deg8, w):
    dout = w.shape[1]
    return pl.pallas_call(
        _m1_body,
        grid=(_TG, _N // _NB),
        in_specs=[
            pl.BlockSpec((1, _NB, _F), lambda t, j: (t, j, 0)),
            pl.BlockSpec((1, _NB, 8), lambda t, j: (t, j, 0)),
            pl.BlockSpec((_F, dout), lambda t, j: (0, 0)),
        ],
        out_specs=pl.BlockSpec((1, _NB, dout), lambda t, j: (t, j, 0)),
        out_shape=jax.ShapeDtypeStruct((_TG, _N, dout), jnp.float32),
    )(x, deg8, w)


def _mid_body(scale_out, s_ref, zn_ref, d_ref, b_ref, w_ref, o_ref):
    nrm = lax.rsqrt(d_ref[0][:, 0:1] + 1.0)
    h = jnp.maximum((s_ref[0] + zn_ref[0]) * nrm + b_ref[...], 0.0)
    z = jnp.dot(h, w_ref[...], preferred_element_type=jnp.float32)
    o_ref[0] = z * nrm if scale_out else z


def _matmul_mid(s, zn, deg8, b, w, scale_out):
    din = zn.shape[2]
    dout = w.shape[1]
    return pl.pallas_call(
        functools.partial(_mid_body, scale_out),
        grid=(_TG, _N // _NB),
        in_specs=[
            pl.BlockSpec((1, _NB, din), lambda t, j: (t, j, 0)),
            pl.BlockSpec((1, _NB, din), lambda t, j: (t, j, 0)),
            pl.BlockSpec((1, _NB, 8), lambda t, j: (t, j, 0)),
            pl.BlockSpec((1, din), lambda t, j: (0, 0)),
            pl.BlockSpec((din, dout), lambda t, j: (0, 0)),
        ],
        out_specs=pl.BlockSpec((1, _NB, dout), lambda t, j: (t, j, 0)),
        out_shape=jax.ShapeDtypeStruct((_TG, _N, dout), jnp.float32),
    )(s, zn, deg8, b.reshape(1, din), w)


def _routing_body(u_ref, v_ref):
    u = u_ref[0]
    bT = jnp.zeros((1, _N), jnp.float32)
    v = jnp.zeros((1, _C), jnp.float32)
    for _ in range(3):
        m = jnp.max(bT)
        e = jnp.exp(bT - m)
        cT = e / jnp.sum(e)
        sv = jnp.dot(cT, u, preferred_element_type=jnp.float32)
        n2 = jnp.sum(sv * sv, axis=1, keepdims=True)
        v = (n2 / (1.0 + n2)) * sv / jnp.sqrt(n2 + 1e-9)
        bT = bT + lax.dot_general(v, u, (((1,), (1,)), ((), ())),
                                  preferred_element_type=jnp.float32)
    v_ref[0] = v


def _routing(u):
    return pl.pallas_call(
        _routing_body,
        grid=(_TG,),
        in_specs=[pl.BlockSpec((1, _N, _C), lambda t: (t, 0, 0))],
        out_specs=pl.BlockSpec((1, 1, _C), lambda t: (t, 0, 0)),
        out_shape=jax.ShapeDtypeStruct((_TG, 1, _C), jnp.float32),
    )(u)


def _lstm_body(v_ref, wi1_ref, wh1_ref, b1_ref, wi2_ref, wh2_ref, b2_ref,
               wd1_ref, bd1_ref, wd2_ref, bd2_ref, g_ref, p_ref):
    h1 = jnp.zeros((1, _H), jnp.float32)
    c1 = jnp.zeros((1, _H), jnp.float32)
    h2 = jnp.zeros((1, _H), jnp.float32)
    c2 = jnp.zeros((1, _H), jnp.float32)
    for t in range(_T):
        x = v_ref[t:t + 1, :]
        z = (jnp.dot(x, wi1_ref[...], preferred_element_type=jnp.float32)
             + jnp.dot(h1, wh1_ref[...], preferred_element_type=jnp.float32)
             + b1_ref[...])
        i = jax.nn.sigmoid(z[:, :_H])
        f = jax.nn.sigmoid(z[:, _H:2 * _H])
        g = jnp.tanh(z[:, 2 * _H:3 * _H])
        o = jax.nn.sigmoid(z[:, 3 * _H:])
        c1 = f * c1 + i * g
        h1 = o * jnp.tanh(c1)
        z2 = (jnp.dot(h1, wi2_ref[...], preferred_element_type=jnp.float32)
              + jnp.dot(h2, wh2_ref[...], preferred_element_type=jnp.float32)
              + b2_ref[...])
        i2 = jax.nn.sigmoid(z2[:, :_H])
        f2 = jax.nn.sigmoid(z2[:, _H:2 * _H])
        g2 = jnp.tanh(z2[:, 2 * _H:3 * _H])
        o2 = jax.nn.sigmoid(z2[:, 3 * _H:])
        c2 = f2 * c2 + i2 * g2
        h2 = o2 * jnp.tanh(c2)
    g_ref[...] = h2
    hd = jnp.maximum(
        jnp.dot(h2, wd1_ref[...], preferred_element_type=jnp.float32)
        + bd1_ref[...], 0.0)
    p = jnp.maximum(
        jnp.dot(hd, wd2_ref[...], preferred_element_type=jnp.float32)
        + bd2_ref[...], 0.0)
    p_ref[...] = p


def _lstm_head(v, Wi1, Wh1, b1s, Wi2, Wh2, b2s, Wd1, bd1, Wd2, bd2):
    return pl.pallas_call(
        _lstm_body,
        out_shape=(jax.ShapeDtypeStruct((1, _H), jnp.float32),
                   jax.ShapeDtypeStruct((1, 1), jnp.float32)),
    )(v, Wi1, Wh1, b1s, Wi2, Wh2, b2s, Wd1, bd1, Wd2, bd2)


def kernel(features, edges, W1, b1, W2, b2, W3, b3, Wc, Wi1, Wh1, bi1, bh1,
           Wi2, Wh2, bi2, bh2, Wd1, bd1, Wd2, bd2):
    src = edges[:, 0, :]
    dst = edges[:, 1, :]
    pad = _EP - _E
    srcp = jnp.pad(src, ((0, 0), (0, pad)))
    dstp = jnp.pad(dst, ((0, 0), (0, pad)), constant_values=_N)
    srcg = srcp + ((jnp.arange(_T, dtype=jnp.int32) % _TG) * _N)[:, None]
    srcg = srcg.reshape(_T, _NS, _CH, _CHK)
    dstp = dstp.reshape(_T, _NS, _CH, _CHK)

    ones8 = jnp.ones((_CHK, 8), jnp.float32)
    zeros8 = jnp.zeros((_ZR, 8), jnp.float32)
    zeros64 = jnp.zeros((_ZR, _F1), jnp.float32)
    zeros32 = jnp.zeros((_ZR, _C), jnp.float32)

    ngrp = _T // _TG
    sg = [srcg[h * _TG:(h + 1) * _TG] for h in range(ngrp)]
    dg = [dstp[h * _TG:(h + 1) * _TG] for h in range(ngrp)]
    fg = [features[h * _TG:(h + 1) * _TG] for h in range(ngrp)]

    deg8 = [_make_deg_kernel()(dg[h], ones8, zeros8) for h in range(ngrp)]

    zn1, s1, zn2, s2, zn3, s3, u, v = ([None] * ngrp for _ in range(8))
    for h in range(ngrp):
        zn1[h] = _matmul1(fg[h], deg8[h], W1)
        s1[h] = _make_agg_kernel(_F1, 8, 4)(
            zn1[h].reshape(_TG * _N, _F1), sg[h], dg[h], zeros64)
    for h in range(ngrp):
        zn2[h] = _matmul_mid(s1[h], zn1[h], deg8[h], b1, W2, True)
        s2[h] = _make_agg_kernel(_C, 8, 4)(
            zn2[h].reshape(_TG * _N, _C), sg[h], dg[h], zeros32)
    for h in range(ngrp):
        zn3[h] = _matmul_mid(s2[h], zn2[h], deg8[h], b2, W3, True)
        s3[h] = _make_agg_kernel(_C, 8, 4)(
            zn3[h].reshape(_TG * _N, _C), sg[h], dg[h], zeros32)
    for h in range(ngrp):
        u[h] = _matmul_mid(s3[h], zn3[h], deg8[h], b3, Wc, False)
        v[h] = _routing(u[h])

    v = jnp.concatenate(v).reshape(_T, _C)

    graph_rep, p = _lstm_head(
        v, Wi1, Wh1, (bi1 + bh1).reshape(1, 4 * _H),
        Wi2, Wh2, (bi2 + bh2).reshape(1, 4 * _H),
        Wd1, bd1.reshape(1, 32), Wd2, bd2.reshape(1, 1))
    return graph_rep, p.reshape((1,))
